# baseline (speedup 1.0000x reference)
import functools

import jax
import jax.numpy as jnp
from jax import lax
from jax.experimental import pallas as pl
from jax.experimental.pallas import tpu as pltpu
from jax.experimental.pallas import tpu_sc as plsc

NC = 2
NS = 16
NW = NC * NS
CHUNK = 128


def _npad(n):
    return ((n + NS * 8 - 1) // (NS * 8)) * (NS * 8)


def _pad_edges(src, dst, n):
    e = src.shape[0]
    sink = _npad(n) - n
    if e % NW:
        head = NW - e % NW
        src = jnp.concatenate([src, jnp.zeros((head,), jnp.int32)])
        dst = jnp.concatenate(
            [dst, (jnp.arange(head, dtype=jnp.int32) % max(sink, 1)) + n])
        e += head
    et = e // NS
    assert et % 16 == 0
    steps0 = et // CHUNK
    steps = steps0 - ((steps0 - 2) % 4)
    rem_w = et - steps * CHUNK
    assert steps >= 2 and rem_w % 16 == 0 and rem_w < 3 * CHUNK
    pairs = steps // 2
    sw = src.reshape(NS, et)
    dw = dst.reshape(NS, et)
    pad_row = jnp.zeros((1, 2, CHUNK), jnp.int32)
    main_s = jnp.concatenate(
        [sw[:, :steps * CHUNK].reshape(NS * pairs, 2, CHUNK), pad_row])
    main_d = jnp.concatenate(
        [dw[:, :steps * CHUNK].reshape(NS * pairs, 2, CHUNK), pad_row])
    rem_s = sw[:, steps * CHUNK:].reshape(-1)
    rem_d = dw[:, steps * CHUNK:].reshape(-1)
    ew = e // NW
    steps_d = (ew // CHUNK) & ~1
    tail_d = ew - steps_d * CHUNK
    assert tail_d <= CHUNK
    dst1 = jnp.concatenate([dst, jnp.zeros((CHUNK,), jnp.int32)])
    return (main_s, main_d, rem_s, rem_d, pairs, rem_w,
            dst1, ew, steps_d, tail_d, 0)

_mesh = lambda: plsc.VectorSubcoreMesh(core_axis_name="c", subcore_axis_name="s",
                                       num_cores=NC, num_subcores=NS)


def _zero_fill(zbuf, rows, width):
    def st(i, _):
        r = i // (width // 16)
        k = i % (width // 16)
        zbuf[r, pl.ds(k * 16, 16)] = jnp.zeros((16,), jnp.float32)
        return 0
    lax.fori_loop(0, rows * (width // 16), st, 0)


def _sc_apply(table2, main_s, main_d, rem_s, rem_d, pairs, rem_w, n, width):
    hw = width // 2
    npad = _npad(n)
    rows_t = npad // NS
    zrows = 8
    iters = (pairs - 1) // 2
    assert pairs % 2 == 1 and rows_t % zrows == 0
    r1 = max(rem_w, 16)
    pieces = []
    off = 0
    while off < rem_w:
        pieces.append((off, min(CHUNK, rem_w - off)))
        off += pieces[-1][1]

    @functools.partial(
        pl.kernel,
        out_type=jax.ShapeDtypeStruct((NC, npad, hw), jnp.float32),
        mesh=_mesh(),
        scratch_types=[
            pltpu.VMEM((2, CHUNK), jnp.int32),
            pltpu.VMEM((2, CHUNK), jnp.int32),
            pltpu.VMEM((2, CHUNK), jnp.int32),
            pltpu.VMEM((2, CHUNK), jnp.int32),
            pltpu.VMEM((CHUNK, hw), jnp.float32),
            pltpu.VMEM((CHUNK, hw), jnp.float32),
            pltpu.VMEM((CHUNK, hw), jnp.float32),
            pltpu.VMEM((CHUNK, hw), jnp.float32),
            pltpu.VMEM((r1,), jnp.int32),
            [pltpu.VMEM((p[1],), jnp.int32) for p in pieces] or
            [pltpu.VMEM((16,), jnp.int32)],
            pltpu.VMEM((r1, hw), jnp.float32),
            pltpu.VMEM((zrows, hw), jnp.float32),
            pltpu.VMEM_SHARED((npad, hw), jnp.float32),
        ] + [pltpu.SemaphoreType.DMA] * 8,
        compiler_params=pltpu.CompilerParams(
            use_tc_tiling_on_sc=(hw % 128 == 0)),
    )
    def k(src3_h, dst3_h, rsrc_h, rdst_h, table_hbm, out_hbm,
          srcP, dstP, srcQ, dstQ, rP0, rP1, rQ0, rQ1, srcR, dstRs, rowsR,
          zbuf, acc, gP0, gP1, gQ0, gQ1, sP0, sP1, sQ0, sQ1):
        c = lax.axis_index("c")
        s = lax.axis_index("s")
        prow = s * pairs
        bias = c * n

        def idxp(sb, db, p):
            pltpu.sync_copy(src3_h.at[prow + p], sb)
            pltpu.sync_copy(dst3_h.at[prow + p], db)
            for h in range(2):
                for j in range(CHUNK // 16):
                    sl = (h, pl.ds(j * 16, 16))
                    sb[sl] = sb[sl] + bias

        def gst(sb, h, rb, sem):
            pltpu.async_copy(table_hbm.at[sb.at[h]], rb, sem)

        def gwt(sb, h, rb, sem):
            pltpu.make_async_copy(table_hbm.at[sb.at[h]], rb, sem).wait()

        def sst(db, h, rb, sem):
            pltpu.async_copy(rb, acc.at[db.at[h]], sem, add=True)

        def swt(db, h, rb, sem):
            pltpu.make_async_copy(rb, acc.at[db.at[h]], sem).wait()

        _zero_fill(zbuf, zrows, hw)

        def zc(i, _):
            pltpu.sync_copy(zbuf, acc.at[pl.ds(s * rows_t + i * zrows, zrows)])
            return 0
        lax.fori_loop(0, rows_t // zrows, zc, 0)

        idxp(srcP, dstP, 0)
        gst(srcP, 0, rP0, gP0)
        gst(srcP, 1, rP1, gP1)
        plsc.subcore_barrier()

        idxp(srcQ, dstQ, 1)
        gst(srcQ, 0, rQ0, gQ0)
        gst(srcQ, 1, rQ1, gQ1)
        gwt(srcP, 0, rP0, gP0)
        sst(dstP, 0, rP0, sP0)
        gwt(srcP, 1, rP1, gP1)
        sst(dstP, 1, rP1, sP1)

        def body(i, _):
            p = 2 * i + 1
            swt(dstP, 0, rP0, sP0)
            swt(dstP, 1, rP1, sP1)
            idxp(srcP, dstP, p + 1)
            gst(srcP, 0, rP0, gP0)
            gst(srcP, 1, rP1, gP1)
            gwt(srcQ, 0, rQ0, gQ0)
            sst(dstQ, 0, rQ0, sQ0)
            gwt(srcQ, 1, rQ1, gQ1)
            sst(dstQ, 1, rQ1, sQ1)
            swt(dstQ, 0, rQ0, sQ0)
            swt(dstQ, 1, rQ1, sQ1)
            idxp(srcQ, dstQ, p + 2)
            gst(srcQ, 0, rQ0, gQ0)
            gst(srcQ, 1, rQ1, gQ1)
            gwt(srcP, 0, rP0, gP0)
            sst(dstP, 0, rP0, sP0)
            gwt(srcP, 1, rP1, gP1)
            sst(dstP, 1, rP1, sP1)
            return 0
        lax.fori_loop(0, iters, body, 0)

        gwt(srcQ, 0, rQ0, gQ0)
        gwt(srcQ, 1, rQ1, gQ1)
        swt(dstP, 0, rP0, sP0)
        swt(dstP, 1, rP1, sP1)

        if rem_w:
            bR = pl.multiple_of(s * rem_w, 8)
            pltpu.sync_copy(rsrc_h.at[pl.ds(bR, rem_w)], srcR)
            for j in range(rem_w // 16):
                sl = (pl.ds(j * 16, 16),)
                srcR[sl] = srcR[sl] + bias
            for kk, (po, sz) in enumerate(pieces):
                pltpu.sync_copy(rdst_h.at[pl.ds(bR + po, sz)], dstRs[kk])
                pltpu.async_copy(table_hbm.at[srcR.at[pl.ds(po, sz)]],
                                 rowsR.at[pl.ds(0, sz)], gP0).wait()
                pltpu.sync_copy(rowsR.at[pl.ds(0, sz)], acc.at[dstRs[kk]],
                                add=True)
        plsc.subcore_barrier()

        pltpu.sync_copy(acc.at[pl.ds(s * rows_t, rows_t)],
                        out_hbm.at[c, pl.ds(s * rows_t, rows_t)])

    return k(main_s, main_d, rem_s, rem_d, table2)[:, :n]


def _sc_degree(dst, ew, steps, rem, rem2, n):
    npad = _npad(n)
    rows_t = npad // NS
    zrows = 8
    width = 16
    r1 = max(rem, 8)
    r2 = max(rem2, 8)

    @functools.partial(
        pl.kernel,
        out_type=jax.ShapeDtypeStruct((NC, npad, width), jnp.float32),
        mesh=_mesh(),
        scratch_types=[
            pltpu.VMEM((CHUNK,), jnp.int32),
            pltpu.VMEM((CHUNK,), jnp.int32),
            pltpu.VMEM((r1,), jnp.int32),
            pltpu.VMEM((r2,), jnp.int32),
            pltpu.VMEM((CHUNK, width), jnp.float32),
            pltpu.VMEM((zrows, width), jnp.float32),
            pltpu.VMEM_SHARED((npad, width), jnp.float32),
            pltpu.SemaphoreType.DMA,
            pltpu.SemaphoreType.DMA,
        ],
        compiler_params=pltpu.CompilerParams(use_tc_tiling_on_sc=False),
    )
    def k(dst_hbm, out_hbm, dstA, dstB, dstR, dstR2, ones_v, zbuf, acc,
          isemA, isemB):
        c = lax.axis_index("c")
        s = lax.axis_index("s")
        wid = s * NC + c

        _zero_fill(zbuf, zrows, width)

        def of(i, _):
            ones_v[i, pl.ds(0, 16)] = jnp.ones((16,), jnp.float32)
            return 0
        lax.fori_loop(0, CHUNK, of, 0)

        def zc(i, _):
            pltpu.sync_copy(zbuf, acc.at[pl.ds(s * rows_t + i * zrows, zrows)])
            return 0
        lax.fori_loop(0, rows_t // zrows, zc, 0)

        base0 = pl.multiple_of(wid * ew, 8)
        pltpu.async_copy(dst_hbm.at[pl.ds(base0, CHUNK)], dstA, isemA)
        plsc.subcore_barrier()

        def body(i, _):
            j0 = 2 * i
            b1 = pl.multiple_of(wid * ew + (j0 + 1) * CHUNK, 8)
            pltpu.async_copy(dst_hbm.at[pl.ds(b1, CHUNK)], dstB, isemB)
            pltpu.make_async_copy(dst_hbm.at[pl.ds(b1, CHUNK)], dstA,
                                  isemA).wait()
            pltpu.sync_copy(ones_v, acc.at[dstA], add=True)
            b2 = pl.multiple_of(wid * ew + (j0 + 2) * CHUNK, 8)
            pltpu.async_copy(dst_hbm.at[pl.ds(b2, CHUNK)], dstA, isemA)
            pltpu.make_async_copy(dst_hbm.at[pl.ds(b2, CHUNK)], dstB,
                                  isemB).wait()
            pltpu.sync_copy(ones_v, acc.at[dstB], add=True)
            return 0
        lax.fori_loop(0, steps // 2, body, 0)
        pltpu.make_async_copy(dst_hbm.at[pl.ds(base0, CHUNK)], dstA,
                              isemA).wait()

        if rem:
            bR = pl.multiple_of(wid * ew + steps * CHUNK, 8)
            pltpu.sync_copy(dst_hbm.at[pl.ds(bR, rem)], dstR)
            pltpu.sync_copy(ones_v.at[pl.ds(0, rem)], acc.at[dstR], add=True)
        if rem2:
            bR2 = pl.multiple_of(wid * ew + steps * CHUNK + rem, 8)
            pltpu.sync_copy(dst_hbm.at[pl.ds(bR2, rem2)], dstR2)
            pltpu.sync_copy(ones_v.at[pl.ds(0, rem2)], acc.at[dstR2], add=True)
        plsc.subcore_barrier()

        pltpu.sync_copy(acc.at[pl.ds(s * rows_t, rows_t)],
                        out_hbm.at[c, pl.ds(s * rows_t, rows_t)])

    return k(dst)[:, :n]


_RB = 2000


def _row_spec(width):
    return pl.BlockSpec((_RB, width), lambda i: (i, 0))


def _part_spec(width):
    return pl.BlockSpec((NC, _RB, width), lambda i: (0, i, 0))


def _full_spec(shape):
    return pl.BlockSpec(shape, lambda i: tuple(0 for _ in shape))


def _cat(ref):
    return jnp.concatenate([ref[0], ref[1]], axis=1)


def _split(arr, ref):
    hw = arr.shape[1] // 2
    ref[0] = arr[:, :hw]
    ref[1] = arr[:, hw:]


def _tc_prep(degp, x, Wv1):
    n, d = x.shape
    h = Wv1.shape[1]

    def body(degp_ref, x_ref, wv_ref, dinv_ref, xs_ref, hv1_ref):
        deg = degp_ref[0, :, 0:1] + degp_ref[1, :, 0:1] + 1.0
        dinv = lax.rsqrt(deg)
        dinv_ref[...] = dinv
        _split(x_ref[...] * dinv, xs_ref)
        _split(dinv * jnp.dot(x_ref[...], wv_ref[...],
                              preferred_element_type=jnp.float32), hv1_ref)

    return pl.pallas_call(
        body,
        grid=(n // _RB,),
        in_specs=[_part_spec(16), _row_spec(d), _full_spec((d, h))],
        out_specs=[_row_spec(1), _part_spec(d // 2), _part_spec(h // 2)],
        out_shape=[jax.ShapeDtypeStruct((n, 1), jnp.float32),
                   jax.ShapeDtypeStruct((2, n, d // 2), jnp.float32),
                   jax.ShapeDtypeStruct((2, n, h // 2), jnp.float32)],
    )(degp, x, Wv1)


def _tc_layer1(p, pv1, xs2, hv12, dinv, Wp1, bp1, bv1, Wv2):
    d = 2 * xs2.shape[2]
    n = xs2.shape[1]
    h = Wp1.shape[1]

    def body(p_ref, pv1_ref, xs_ref, hv1_ref, dinv_ref, wp_ref, bp_ref,
             bv_ref, wv2_ref, xa_ref, hv2_ref):
        dv = dinv_ref[...]
        z = dv * (_cat(p_ref) + _cat(xs_ref))
        a1 = jnp.maximum(jnp.dot(z, wp_ref[...],
                                 preferred_element_type=jnp.float32)
                         + bp_ref[...], 0.0)
        _split(dv * a1, xa_ref)
        v1 = jnp.maximum(dv * (_cat(pv1_ref) + _cat(hv1_ref))
                         + bv_ref[...], 0.0)
        _split(dv * jnp.dot(v1, wv2_ref[...],
                            preferred_element_type=jnp.float32), hv2_ref)

    return pl.pallas_call(
        body,
        grid=(n // _RB,),
        in_specs=[_part_spec(d // 2), _part_spec(h // 2), _part_spec(d // 2),
                  _part_spec(h // 2), _row_spec(1),
                  _full_spec((d, h)), _full_spec((1, h)), _full_spec((1, h)),
                  _full_spec((h, h))],
        out_specs=[_part_spec(h // 2), _part_spec(h // 2)],
        out_shape=[jax.ShapeDtypeStruct((2, n, h // 2), jnp.float32),
                   jax.ShapeDtypeStruct((2, n, h // 2), jnp.float32)],
    )(p, pv1, xs2, hv12, dinv, Wp1, bp1.reshape(1, -1), bv1.reshape(1, -1),
      Wv2)


def _tc_layer23(pa, pv2, xa2, hv22, dinv, Wp2, bp2, Wp3, bv2, Wv3):
    n = xa2.shape[1]
    h = 2 * xa2.shape[2]
    out_p = Wp3.shape[1]

    def body(pa_ref, pv2_ref, xa_ref, hv2_ref, dinv_ref,
             wp2_ref, bp2_ref, wp3_ref, bv2_ref, wv3_ref, hcat_ref):
        dv = dinv_ref[...]
        za = dv * (_cat(pa_ref) + _cat(xa_ref))
        a2 = jnp.maximum(jnp.dot(za, wp2_ref[...],
                                 preferred_element_type=jnp.float32)
                         + bp2_ref[...], 0.0)
        hp = jnp.dot(a2, wp3_ref[...], preferred_element_type=jnp.float32)
        v2 = jnp.maximum(dv * (_cat(pv2_ref) + _cat(hv2_ref))
                         + bv2_ref[...], 0.0)
        hv = jnp.dot(v2, wv3_ref[...], preferred_element_type=jnp.float32)
        if out_p == 16:
            hcat_ref[0] = dv * hp
        else:
            pad = jnp.zeros((hp.shape[0], 16 - out_p), jnp.float32)
            hcat_ref[0] = dv * jnp.concatenate([hp, pad], axis=1)
        pad2 = jnp.zeros((hp.shape[0], 15), jnp.float32)
        hcat_ref[1] = dv * jnp.concatenate([hv, pad2], axis=1)

    return pl.pallas_call(
        body,
        grid=(n // _RB,),
        in_specs=[_part_spec(h // 2), _part_spec(h // 2), _part_spec(h // 2),
                  _part_spec(h // 2), _row_spec(1),
                  _full_spec((h, h)), _full_spec((1, h)),
                  _full_spec((h, out_p)),
                  _full_spec((1, h)),
                  _full_spec((h, 1))],
        out_specs=[_part_spec(16)],
        out_shape=[jax.ShapeDtypeStruct((2, n, 16), jnp.float32)],
    )(pa, pv2, xa2, hv22, dinv, Wp2, bp2.reshape(1, -1), Wp3,
      bv2.reshape(1, -1), Wv3)[0]


def _tc_final(pc, hcat2, dinv, bp3, bv3, out_p):
    n = hcat2.shape[1]

    def body(pc_ref, hcat_ref, dinv_ref, bp3_ref, bv3_ref, lg_ref, vl_ref):
        dv = dinv_ref[...]
        lg_ref[...] = (dv * (pc_ref[0] + hcat_ref[0]))[:, :out_p] \
            + bp3_ref[...]
        vl_ref[...] = dv * (pc_ref[1, :, 0:1] + hcat_ref[1, :, 0:1]) \
            + bv3_ref[...]

    return pl.pallas_call(
        body,
        grid=(n // _RB,),
        in_specs=[_part_spec(16), _part_spec(16), _row_spec(1),
                  _full_spec((1, out_p)), _full_spec((1, 1))],
        out_specs=[_row_spec(out_p), _row_spec(1)],
        out_shape=[jax.ShapeDtypeStruct((n, out_p), jnp.float32),
                   jax.ShapeDtypeStruct((n, 1), jnp.float32)],
    )(pc, hcat2, dinv, bp3.reshape(1, -1), bv3.reshape(1, -1))


def kernel(x, edge_index, Wp1, bp1, Wp2, bp2, Wp3, bp3, Wv1, bv1, Wv2, bv2,
           Wv3, bv3):
    n, d = x.shape
    out_p = Wp3.shape[1]
    (main_s, main_d, rem_s, rem_d, pairs, rem_w,
     dst1, ew, steps_d, drem, drem2) = _pad_edges(edge_index[0],
                                                  edge_index[1], n)

    degp = _sc_degree(dst1, ew, steps_d, drem, drem2, n)
    dinv, xs2, hv12 = _tc_prep(degp, x, Wv1)

    def apply2(t2, width):
        return _sc_apply(t2.reshape(2 * n, width // 2), main_s, main_d,
                         rem_s, rem_d, pairs, rem_w, n, width)

    p0 = apply2(xs2, d)
    pv1 = apply2(hv12, d)
    xa2, hv22 = _tc_layer1(p0, pv1, xs2, hv12, dinv, Wp1, bp1, bv1, Wv2)

    pa = apply2(xa2, d)
    pv2 = apply2(hv22, d)
    hcat2 = _tc_layer23(pa, pv2, xa2, hv22, dinv, Wp2, bp2, Wp3, bv2, Wv3)

    pc = apply2(hcat2, 32)
    logits, value = _tc_final(pc, hcat2, dinv, bp3, bv3, out_p)
    return (logits, value)


# speedup vs baseline: 1.0053x; 1.0053x over previous
import functools

import jax
import jax.numpy as jnp
from jax import lax
from jax.experimental import pallas as pl
from jax.experimental.pallas import tpu as pltpu
from jax.experimental.pallas import tpu_sc as plsc

NC = 2
NS = 16
NW = NC * NS
CHUNK = 128


def _npad(n):
    return ((n + NS * 8 - 1) // (NS * 8)) * (NS * 8)


def _pad_edges(src, dst, n):
    e = src.shape[0]
    sink = _npad(n) - n
    if e % NW:
        head = NW - e % NW
        src = jnp.concatenate([src, jnp.zeros((head,), jnp.int32)])
        dst = jnp.concatenate(
            [dst, (jnp.arange(head, dtype=jnp.int32) % max(sink, 1)) + n])
        e += head
    et = e // NS
    assert et % 16 == 0
    steps0 = et // CHUNK
    steps = steps0 - ((steps0 - 2) % 4)
    rem_w = et - steps * CHUNK
    assert steps >= 2 and rem_w % 16 == 0 and rem_w < 3 * CHUNK
    pairs = steps // 2
    sw = src.reshape(NS, et)
    dw = dst.reshape(NS, et)
    pad_row = jnp.zeros((1, 2, CHUNK), jnp.int32)
    main_s = jnp.concatenate(
        [sw[:, :steps * CHUNK].reshape(NS * pairs, 2, CHUNK), pad_row])
    main_d = jnp.concatenate(
        [dw[:, :steps * CHUNK].reshape(NS * pairs, 2, CHUNK), pad_row])
    rem_s = sw[:, steps * CHUNK:].reshape(-1)
    rem_d = dw[:, steps * CHUNK:].reshape(-1)
    ew = e // NW
    steps_d = (ew // CHUNK) & ~1
    tail_d = ew - steps_d * CHUNK
    assert tail_d <= CHUNK
    dst1 = jnp.concatenate([dst, jnp.zeros((CHUNK,), jnp.int32)])
    return (main_s, main_d, rem_s, rem_d, pairs, rem_w,
            dst1, ew, steps_d, tail_d, 0)

_mesh = lambda: plsc.VectorSubcoreMesh(core_axis_name="c", subcore_axis_name="s",
                                       num_cores=NC, num_subcores=NS)


def _zero_fill(zbuf, rows, width):
    def st(i, _):
        r = i // (width // 16)
        k = i % (width // 16)
        zbuf[r, pl.ds(k * 16, 16)] = jnp.zeros((16,), jnp.float32)
        return 0
    lax.fori_loop(0, rows * (width // 16), st, 0)


def _sc_apply(table2, main_s, main_d, rem_s, rem_d, pairs, rem_w, n, width):
    hw = width // 2
    npad = _npad(n)
    rows_t = npad // NS
    zrows = 8
    iters = (pairs - 1) // 2
    assert pairs % 2 == 1 and rows_t % zrows == 0
    r1 = max(rem_w, 16)
    pieces = []
    off = 0
    while off < rem_w:
        pieces.append((off, min(CHUNK, rem_w - off)))
        off += pieces[-1][1]

    @functools.partial(
        pl.kernel,
        out_type=jax.ShapeDtypeStruct((NC, npad, hw), jnp.float32),
        mesh=_mesh(),
        scratch_types=[
            pltpu.VMEM((2, CHUNK), jnp.int32),
            pltpu.VMEM((2, CHUNK), jnp.int32),
            pltpu.VMEM((2, CHUNK), jnp.int32),
            pltpu.VMEM((2, CHUNK), jnp.int32),
            pltpu.VMEM((CHUNK, hw), jnp.float32),
            pltpu.VMEM((CHUNK, hw), jnp.float32),
            pltpu.VMEM((CHUNK, hw), jnp.float32),
            pltpu.VMEM((CHUNK, hw), jnp.float32),
            pltpu.VMEM((r1,), jnp.int32),
            [pltpu.VMEM((p[1],), jnp.int32) for p in pieces] or
            [pltpu.VMEM((16,), jnp.int32)],
            pltpu.VMEM((r1, hw), jnp.float32),
            pltpu.VMEM((zrows, hw), jnp.float32),
            pltpu.VMEM_SHARED((npad, hw), jnp.float32),
        ] + [pltpu.SemaphoreType.DMA] * 8,
        compiler_params=pltpu.CompilerParams(
            use_tc_tiling_on_sc=(hw % 128 == 0)),
    )
    def k(src3_h, dst3_h, rsrc_h, rdst_h, table_hbm, out_hbm,
          srcP, dstP, srcQ, dstQ, rP0, rP1, rQ0, rQ1, srcR, dstRs, rowsR,
          zbuf, acc, gP0, gP1, gQ0, gQ1, sP0, sP1, sQ0, sQ1):
        c = lax.axis_index("c")
        s = lax.axis_index("s")
        prow = s * pairs

        def idxp(sb, db, p):
            pltpu.sync_copy(src3_h.at[prow + p], sb)
            pltpu.sync_copy(dst3_h.at[prow + p], db)

        def gst(sb, h, rb, sem):
            pltpu.async_copy(table_hbm.at[c].at[sb.at[h]], rb, sem)

        def gwt(sb, h, rb, sem):
            pltpu.make_async_copy(table_hbm.at[c].at[sb.at[h]], rb, sem).wait()

        def sst(db, h, rb, sem):
            pltpu.async_copy(rb, acc.at[db.at[h]], sem, add=True)

        def swt(db, h, rb, sem):
            pltpu.make_async_copy(rb, acc.at[db.at[h]], sem).wait()

        _zero_fill(zbuf, zrows, hw)

        def zc(i, _):
            pltpu.sync_copy(zbuf, acc.at[pl.ds(s * rows_t + i * zrows, zrows)])
            return 0
        lax.fori_loop(0, rows_t // zrows, zc, 0)

        idxp(srcP, dstP, 0)
        gst(srcP, 0, rP0, gP0)
        gst(srcP, 1, rP1, gP1)
        plsc.subcore_barrier()

        idxp(srcQ, dstQ, 1)
        gst(srcQ, 0, rQ0, gQ0)
        gst(srcQ, 1, rQ1, gQ1)
        gwt(srcP, 0, rP0, gP0)
        sst(dstP, 0, rP0, sP0)
        gwt(srcP, 1, rP1, gP1)
        sst(dstP, 1, rP1, sP1)

        def body(i, _):
            p = 2 * i + 1
            swt(dstP, 0, rP0, sP0)
            swt(dstP, 1, rP1, sP1)
            idxp(srcP, dstP, p + 1)
            gst(srcP, 0, rP0, gP0)
            gst(srcP, 1, rP1, gP1)
            gwt(srcQ, 0, rQ0, gQ0)
            sst(dstQ, 0, rQ0, sQ0)
            gwt(srcQ, 1, rQ1, gQ1)
            sst(dstQ, 1, rQ1, sQ1)
            swt(dstQ, 0, rQ0, sQ0)
            swt(dstQ, 1, rQ1, sQ1)
            idxp(srcQ, dstQ, p + 2)
            gst(srcQ, 0, rQ0, gQ0)
            gst(srcQ, 1, rQ1, gQ1)
            gwt(srcP, 0, rP0, gP0)
            sst(dstP, 0, rP0, sP0)
            gwt(srcP, 1, rP1, gP1)
            sst(dstP, 1, rP1, sP1)
            return 0
        lax.fori_loop(0, iters, body, 0)

        gwt(srcQ, 0, rQ0, gQ0)
        gwt(srcQ, 1, rQ1, gQ1)
        swt(dstP, 0, rP0, sP0)
        swt(dstP, 1, rP1, sP1)

        if rem_w:
            bR = pl.multiple_of(s * rem_w, 8)
            pltpu.sync_copy(rsrc_h.at[pl.ds(bR, rem_w)], srcR)
            for kk, (po, sz) in enumerate(pieces):
                pltpu.sync_copy(rdst_h.at[pl.ds(bR + po, sz)], dstRs[kk])
                pltpu.async_copy(table_hbm.at[c].at[srcR.at[pl.ds(po, sz)]],
                                 rowsR.at[pl.ds(0, sz)], gP0).wait()
                pltpu.sync_copy(rowsR.at[pl.ds(0, sz)], acc.at[dstRs[kk]],
                                add=True)
        plsc.subcore_barrier()

        pltpu.sync_copy(acc.at[pl.ds(s * rows_t, rows_t)],
                        out_hbm.at[c, pl.ds(s * rows_t, rows_t)])

    return k(main_s, main_d, rem_s, rem_d, table2)[:, :n]


def _sc_degree(dst, ew, steps, rem, rem2, n):
    npad = _npad(n)
    rows_t = npad // NS
    zrows = 8
    width = 16
    r1 = max(rem, 8)
    r2 = max(rem2, 8)

    @functools.partial(
        pl.kernel,
        out_type=jax.ShapeDtypeStruct((NC, npad, width), jnp.float32),
        mesh=_mesh(),
        scratch_types=[
            pltpu.VMEM((CHUNK,), jnp.int32),
            pltpu.VMEM((CHUNK,), jnp.int32),
            pltpu.VMEM((r1,), jnp.int32),
            pltpu.VMEM((r2,), jnp.int32),
            pltpu.VMEM((CHUNK, width), jnp.float32),
            pltpu.VMEM((zrows, width), jnp.float32),
            pltpu.VMEM_SHARED((npad, width), jnp.float32),
            pltpu.SemaphoreType.DMA,
            pltpu.SemaphoreType.DMA,
        ],
        compiler_params=pltpu.CompilerParams(use_tc_tiling_on_sc=False),
    )
    def k(dst_hbm, out_hbm, dstA, dstB, dstR, dstR2, ones_v, zbuf, acc,
          isemA, isemB):
        c = lax.axis_index("c")
        s = lax.axis_index("s")
        wid = s * NC + c

        _zero_fill(zbuf, zrows, width)

        def of(i, _):
            ones_v[i, pl.ds(0, 16)] = jnp.ones((16,), jnp.float32)
            return 0
        lax.fori_loop(0, CHUNK, of, 0)

        def zc(i, _):
            pltpu.sync_copy(zbuf, acc.at[pl.ds(s * rows_t + i * zrows, zrows)])
            return 0
        lax.fori_loop(0, rows_t // zrows, zc, 0)

        base0 = pl.multiple_of(wid * ew, 8)
        pltpu.async_copy(dst_hbm.at[pl.ds(base0, CHUNK)], dstA, isemA)
        plsc.subcore_barrier()

        def body(i, _):
            j0 = 2 * i
            b1 = pl.multiple_of(wid * ew + (j0 + 1) * CHUNK, 8)
            pltpu.async_copy(dst_hbm.at[pl.ds(b1, CHUNK)], dstB, isemB)
            pltpu.make_async_copy(dst_hbm.at[pl.ds(b1, CHUNK)], dstA,
                                  isemA).wait()
            pltpu.sync_copy(ones_v, acc.at[dstA], add=True)
            b2 = pl.multiple_of(wid * ew + (j0 + 2) * CHUNK, 8)
            pltpu.async_copy(dst_hbm.at[pl.ds(b2, CHUNK)], dstA, isemA)
            pltpu.make_async_copy(dst_hbm.at[pl.ds(b2, CHUNK)], dstB,
                                  isemB).wait()
            pltpu.sync_copy(ones_v, acc.at[dstB], add=True)
            return 0
        lax.fori_loop(0, steps // 2, body, 0)
        pltpu.make_async_copy(dst_hbm.at[pl.ds(base0, CHUNK)], dstA,
                              isemA).wait()

        if rem:
            bR = pl.multiple_of(wid * ew + steps * CHUNK, 8)
            pltpu.sync_copy(dst_hbm.at[pl.ds(bR, rem)], dstR)
            pltpu.sync_copy(ones_v.at[pl.ds(0, rem)], acc.at[dstR], add=True)
        if rem2:
            bR2 = pl.multiple_of(wid * ew + steps * CHUNK + rem, 8)
            pltpu.sync_copy(dst_hbm.at[pl.ds(bR2, rem2)], dstR2)
            pltpu.sync_copy(ones_v.at[pl.ds(0, rem2)], acc.at[dstR2], add=True)
        plsc.subcore_barrier()

        pltpu.sync_copy(acc.at[pl.ds(s * rows_t, rows_t)],
                        out_hbm.at[c, pl.ds(s * rows_t, rows_t)])

    return k(dst)[:, :n]


_RB = 2000


def _row_spec(width):
    return pl.BlockSpec((_RB, width), lambda i: (i, 0))


def _part_spec(width):
    return pl.BlockSpec((NC, _RB, width), lambda i: (0, i, 0))


def _full_spec(shape):
    return pl.BlockSpec(shape, lambda i: tuple(0 for _ in shape))


def _cat(ref):
    return jnp.concatenate([ref[0], ref[1]], axis=1)


def _split(arr, ref):
    hw = arr.shape[1] // 2
    ref[0] = arr[:, :hw]
    ref[1] = arr[:, hw:]


def _tc_prep(degp, x, Wv1):
    n, d = x.shape
    h = Wv1.shape[1]

    def body(degp_ref, x_ref, wv_ref, dinv_ref, xs_ref, hv1_ref):
        deg = degp_ref[0, :, 0:1] + degp_ref[1, :, 0:1] + 1.0
        dinv = lax.rsqrt(deg)
        dinv_ref[...] = dinv
        _split(x_ref[...] * dinv, xs_ref)
        _split(dinv * jnp.dot(x_ref[...], wv_ref[...],
                              preferred_element_type=jnp.float32), hv1_ref)

    return pl.pallas_call(
        body,
        grid=(n // _RB,),
        in_specs=[_part_spec(16), _row_spec(d), _full_spec((d, h))],
        out_specs=[_row_spec(1), _part_spec(d // 2), _part_spec(h // 2)],
        out_shape=[jax.ShapeDtypeStruct((n, 1), jnp.float32),
                   jax.ShapeDtypeStruct((2, n, d // 2), jnp.float32),
                   jax.ShapeDtypeStruct((2, n, h // 2), jnp.float32)],
    )(degp, x, Wv1)


def _tc_layer1(p, pv1, xs2, hv12, dinv, Wp1, bp1, bv1, Wv2):
    d = 2 * xs2.shape[2]
    n = xs2.shape[1]
    h = Wp1.shape[1]

    def body(p_ref, pv1_ref, xs_ref, hv1_ref, dinv_ref, wp_ref, bp_ref,
             bv_ref, wv2_ref, xa_ref, hv2_ref):
        dv = dinv_ref[...]
        z = dv * (_cat(p_ref) + _cat(xs_ref))
        a1 = jnp.maximum(jnp.dot(z, wp_ref[...],
                                 preferred_element_type=jnp.float32)
                         + bp_ref[...], 0.0)
        _split(dv * a1, xa_ref)
        v1 = jnp.maximum(dv * (_cat(pv1_ref) + _cat(hv1_ref))
                         + bv_ref[...], 0.0)
        _split(dv * jnp.dot(v1, wv2_ref[...],
                            preferred_element_type=jnp.float32), hv2_ref)

    return pl.pallas_call(
        body,
        grid=(n // _RB,),
        in_specs=[_part_spec(d // 2), _part_spec(h // 2), _part_spec(d // 2),
                  _part_spec(h // 2), _row_spec(1),
                  _full_spec((d, h)), _full_spec((1, h)), _full_spec((1, h)),
                  _full_spec((h, h))],
        out_specs=[_part_spec(h // 2), _part_spec(h // 2)],
        out_shape=[jax.ShapeDtypeStruct((2, n, h // 2), jnp.float32),
                   jax.ShapeDtypeStruct((2, n, h // 2), jnp.float32)],
    )(p, pv1, xs2, hv12, dinv, Wp1, bp1.reshape(1, -1), bv1.reshape(1, -1),
      Wv2)


def _tc_layer23(pa, pv2, xa2, hv22, dinv, Wp2, bp2, Wp3, bv2, Wv3):
    n = xa2.shape[1]
    h = 2 * xa2.shape[2]
    out_p = Wp3.shape[1]

    def body(pa_ref, pv2_ref, xa_ref, hv2_ref, dinv_ref,
             wp2_ref, bp2_ref, wp3_ref, bv2_ref, wv3_ref, hcat_ref):
        dv = dinv_ref[...]
        za = dv * (_cat(pa_ref) + _cat(xa_ref))
        a2 = jnp.maximum(jnp.dot(za, wp2_ref[...],
                                 preferred_element_type=jnp.float32)
                         + bp2_ref[...], 0.0)
        hp = jnp.dot(a2, wp3_ref[...], preferred_element_type=jnp.float32)
        v2 = jnp.maximum(dv * (_cat(pv2_ref) + _cat(hv2_ref))
                         + bv2_ref[...], 0.0)
        hv = jnp.dot(v2, wv3_ref[...], preferred_element_type=jnp.float32)
        if out_p == 16:
            hcat_ref[0] = dv * hp
        else:
            pad = jnp.zeros((hp.shape[0], 16 - out_p), jnp.float32)
            hcat_ref[0] = dv * jnp.concatenate([hp, pad], axis=1)
        pad2 = jnp.zeros((hp.shape[0], 15), jnp.float32)
        hcat_ref[1] = dv * jnp.concatenate([hv, pad2], axis=1)

    return pl.pallas_call(
        body,
        grid=(n // _RB,),
        in_specs=[_part_spec(h // 2), _part_spec(h // 2), _part_spec(h // 2),
                  _part_spec(h // 2), _row_spec(1),
                  _full_spec((h, h)), _full_spec((1, h)),
                  _full_spec((h, out_p)),
                  _full_spec((1, h)),
                  _full_spec((h, 1))],
        out_specs=[_part_spec(16)],
        out_shape=[jax.ShapeDtypeStruct((2, n, 16), jnp.float32)],
    )(pa, pv2, xa2, hv22, dinv, Wp2, bp2.reshape(1, -1), Wp3,
      bv2.reshape(1, -1), Wv3)[0]


def _tc_final(pc, hcat2, dinv, bp3, bv3, out_p):
    n = hcat2.shape[1]

    def body(pc_ref, hcat_ref, dinv_ref, bp3_ref, bv3_ref, lg_ref, vl_ref):
        dv = dinv_ref[...]
        lg_ref[...] = (dv * (pc_ref[0] + hcat_ref[0]))[:, :out_p] \
            + bp3_ref[...]
        vl_ref[...] = dv * (pc_ref[1, :, 0:1] + hcat_ref[1, :, 0:1]) \
            + bv3_ref[...]

    return pl.pallas_call(
        body,
        grid=(n // _RB,),
        in_specs=[_part_spec(16), _part_spec(16), _row_spec(1),
                  _full_spec((1, out_p)), _full_spec((1, 1))],
        out_specs=[_row_spec(out_p), _row_spec(1)],
        out_shape=[jax.ShapeDtypeStruct((n, out_p), jnp.float32),
                   jax.ShapeDtypeStruct((n, 1), jnp.float32)],
    )(pc, hcat2, dinv, bp3.reshape(1, -1), bv3.reshape(1, -1))


def kernel(x, edge_index, Wp1, bp1, Wp2, bp2, Wp3, bp3, Wv1, bv1, Wv2, bv2,
           Wv3, bv3):
    n, d = x.shape
    out_p = Wp3.shape[1]
    (main_s, main_d, rem_s, rem_d, pairs, rem_w,
     dst1, ew, steps_d, drem, drem2) = _pad_edges(edge_index[0],
                                                  edge_index[1], n)

    degp = _sc_degree(dst1, ew, steps_d, drem, drem2, n)
    dinv, xs2, hv12 = _tc_prep(degp, x, Wv1)

    def apply2(t2, width):
        return _sc_apply(t2, main_s, main_d, rem_s, rem_d, pairs, rem_w, n,
                         width)

    p0 = apply2(xs2, d)
    pv1 = apply2(hv12, d)
    xa2, hv22 = _tc_layer1(p0, pv1, xs2, hv12, dinv, Wp1, bp1, bv1, Wv2)

    pa = apply2(xa2, d)
    pv2 = apply2(hv22, d)
    hcat2 = _tc_layer23(pa, pv2, xa2, hv22, dinv, Wp2, bp2, Wp3, bv2, Wv3)

    pc = apply2(hcat2, 32)
    logits, value = _tc_final(pc, hcat2, dinv, bp3, bv3, out_p)
    return (logits, value)


# speedup vs baseline: 1.1019x; 1.0961x over previous
import functools

import jax
import jax.numpy as jnp
from jax import lax
from jax.experimental import pallas as pl
from jax.experimental.pallas import tpu as pltpu
from jax.experimental.pallas import tpu_sc as plsc

NC = 2
NS = 16
NW = NC * NS
CHUNK = 128

_mesh = lambda: plsc.VectorSubcoreMesh(core_axis_name="c",
                                       subcore_axis_name="s",
                                       num_cores=NC, num_subcores=NS)


def _npad(n):
    return ((n + NS * 8 - 1) // (NS * 8)) * (NS * 8)


def _pad_edges(src, dst, n):
    e = src.shape[0]
    sink = _npad(n) - n
    if e % NW:
        head = NW - e % NW
        src = jnp.concatenate([src, jnp.zeros((head,), jnp.int32)])
        dst = jnp.concatenate(
            [dst, (jnp.arange(head, dtype=jnp.int32) % max(sink, 1)) + n])
        e += head
    ew = e // NW
    assert ew % 8 == 0
    steps = (ew // CHUNK) & ~1
    tail = ew - steps * CHUNK
    rem = min(tail, CHUNK)
    rem2 = tail - rem
    src_p = jnp.concatenate([src, jnp.zeros((CHUNK,), jnp.int32)])
    dst_p = jnp.concatenate([dst, jnp.zeros((CHUNK,), jnp.int32)])
    return src_p, dst_p, ew, steps, rem, rem2


def _zero_fill(zbuf, rows, width):
    def st(i, _):
        r = i // (width // 16)
        k = i % (width // 16)
        zbuf[r, pl.ds(k * 16, 16)] = jnp.zeros((16,), jnp.float32)
        return 0
    lax.fori_loop(0, rows * (width // 16), st, 0)


def _sc_apply(table, src, dst, ew, steps, rem, rem2, n, width):
    npad = _npad(n)
    rows_t = npad // NS
    zrows = 8
    assert steps % 2 == 0 and rows_t % zrows == 0
    r1 = max(rem, 8)
    r2 = max(rem2, 8)

    @functools.partial(
        pl.kernel,
        out_type=jax.ShapeDtypeStruct((NC, npad, width), jnp.float32),
        mesh=_mesh(),
        scratch_types=[
            pltpu.VMEM((CHUNK,), jnp.int32),
            pltpu.VMEM((CHUNK,), jnp.int32),
            pltpu.VMEM((CHUNK,), jnp.int32),
            pltpu.VMEM((CHUNK,), jnp.int32),
            pltpu.VMEM((CHUNK, width), jnp.float32),
            pltpu.VMEM((CHUNK, width), jnp.float32),
            pltpu.VMEM((r1,), jnp.int32),
            pltpu.VMEM((r1,), jnp.int32),
            pltpu.VMEM((r1, width), jnp.float32),
            pltpu.VMEM((r2,), jnp.int32),
            pltpu.VMEM((r2,), jnp.int32),
            pltpu.VMEM((r2, width), jnp.float32),
            pltpu.VMEM((zrows, width), jnp.float32),
            pltpu.VMEM_SHARED((npad, width), jnp.float32),
            pltpu.SemaphoreType.DMA,
            pltpu.SemaphoreType.DMA,
        ],
        compiler_params=pltpu.CompilerParams(
            use_tc_tiling_on_sc=(width % 128 == 0)),
    )
    def k(src_hbm, dst_hbm, table_hbm, out_hbm, srcA, srcB, dstA, dstB,
          rowsA, rowsB, srcR, dstR, rowsR, srcR2, dstR2, rowsR2, zbuf, acc,
          gsemA, gsemB):
        c = lax.axis_index("c")
        s = lax.axis_index("s")
        wid = s * NC + c

        _zero_fill(zbuf, zrows, width)

        def zc(i, _):
            pltpu.sync_copy(zbuf, acc.at[pl.ds(s * rows_t + i * zrows,
                                               zrows)])
            return 0
        lax.fori_loop(0, rows_t // zrows, zc, 0)

        base0 = pl.multiple_of(wid * ew, 8)
        pltpu.sync_copy(src_hbm.at[pl.ds(base0, CHUNK)], srcA)
        pltpu.sync_copy(dst_hbm.at[pl.ds(base0, CHUNK)], dstA)
        pltpu.async_copy(table_hbm.at[srcA], rowsA, gsemA)
        plsc.subcore_barrier()

        def body(i, _):
            j0 = 2 * i
            b1 = pl.multiple_of(wid * ew + (j0 + 1) * CHUNK, 8)
            pltpu.sync_copy(src_hbm.at[pl.ds(b1, CHUNK)], srcB)
            pltpu.sync_copy(dst_hbm.at[pl.ds(b1, CHUNK)], dstB)
            pltpu.async_copy(table_hbm.at[srcB], rowsB, gsemB)
            pltpu.make_async_copy(table_hbm.at[srcA], rowsA, gsemA).wait()
            pltpu.sync_copy(rowsA, acc.at[dstA], add=True)
            b2 = pl.multiple_of(wid * ew + (j0 + 2) * CHUNK, 8)
            pltpu.sync_copy(src_hbm.at[pl.ds(b2, CHUNK)], srcA)
            pltpu.sync_copy(dst_hbm.at[pl.ds(b2, CHUNK)], dstA)
            pltpu.async_copy(table_hbm.at[srcA], rowsA, gsemA)
            pltpu.make_async_copy(table_hbm.at[srcB], rowsB, gsemB).wait()
            pltpu.sync_copy(rowsB, acc.at[dstB], add=True)
            return 0
        lax.fori_loop(0, steps // 2, body, 0)
        pltpu.make_async_copy(table_hbm.at[srcA], rowsA, gsemA).wait()

        if rem:
            bR = pl.multiple_of(wid * ew + steps * CHUNK, 8)
            pltpu.sync_copy(src_hbm.at[pl.ds(bR, rem)], srcR)
            pltpu.sync_copy(dst_hbm.at[pl.ds(bR, rem)], dstR)
            pltpu.async_copy(table_hbm.at[srcR], rowsR, gsemA).wait()
            pltpu.sync_copy(rowsR, acc.at[dstR], add=True)
        if rem2:
            bR2 = pl.multiple_of(wid * ew + steps * CHUNK + rem, 8)
            pltpu.sync_copy(src_hbm.at[pl.ds(bR2, rem2)], srcR2)
            pltpu.sync_copy(dst_hbm.at[pl.ds(bR2, rem2)], dstR2)
            pltpu.async_copy(table_hbm.at[srcR2], rowsR2, gsemB).wait()
            pltpu.sync_copy(rowsR2, acc.at[dstR2], add=True)
        plsc.subcore_barrier()

        pltpu.sync_copy(acc.at[pl.ds(s * rows_t, rows_t)],
                        out_hbm.at[c, pl.ds(s * rows_t, rows_t)])

    return k(src, dst, table)[:, :n]


def _sc_degree(dst, ew, steps, rem, rem2, n):
    npad = _npad(n)
    rows_t = npad // NS
    zrows = 8
    width = 16
    r1 = max(rem, 8)
    r2 = max(rem2, 8)

    @functools.partial(
        pl.kernel,
        out_type=jax.ShapeDtypeStruct((NC, npad, width), jnp.float32),
        mesh=_mesh(),
        scratch_types=[
            pltpu.VMEM((CHUNK,), jnp.int32),
            pltpu.VMEM((CHUNK,), jnp.int32),
            pltpu.VMEM((r1,), jnp.int32),
            pltpu.VMEM((r2,), jnp.int32),
            pltpu.VMEM((CHUNK, width), jnp.float32),
            pltpu.VMEM((zrows, width), jnp.float32),
            pltpu.VMEM_SHARED((npad, width), jnp.float32),
            pltpu.SemaphoreType.DMA,
            pltpu.SemaphoreType.DMA,
        ],
        compiler_params=pltpu.CompilerParams(use_tc_tiling_on_sc=False),
    )
    def k(dst_hbm, out_hbm, dstA, dstB, dstR, dstR2, ones_v, zbuf, acc,
          isemA, isemB):
        c = lax.axis_index("c")
        s = lax.axis_index("s")
        wid = s * NC + c

        _zero_fill(zbuf, zrows, width)

        def of(i, _):
            ones_v[i, pl.ds(0, 16)] = jnp.ones((16,), jnp.float32)
            return 0
        lax.fori_loop(0, CHUNK, of, 0)

        def zc(i, _):
            pltpu.sync_copy(zbuf, acc.at[pl.ds(s * rows_t + i * zrows,
                                               zrows)])
            return 0
        lax.fori_loop(0, rows_t // zrows, zc, 0)

        base0 = pl.multiple_of(wid * ew, 8)
        pltpu.async_copy(dst_hbm.at[pl.ds(base0, CHUNK)], dstA, isemA)
        plsc.subcore_barrier()

        def body(i, _):
            j0 = 2 * i
            b1 = pl.multiple_of(wid * ew + (j0 + 1) * CHUNK, 8)
            pltpu.async_copy(dst_hbm.at[pl.ds(b1, CHUNK)], dstB, isemB)
            pltpu.make_async_copy(dst_hbm.at[pl.ds(b1, CHUNK)], dstA,
                                  isemA).wait()
            pltpu.sync_copy(ones_v, acc.at[dstA], add=True)
            b2 = pl.multiple_of(wid * ew + (j0 + 2) * CHUNK, 8)
            pltpu.async_copy(dst_hbm.at[pl.ds(b2, CHUNK)], dstA, isemA)
            pltpu.make_async_copy(dst_hbm.at[pl.ds(b2, CHUNK)], dstB,
                                  isemB).wait()
            pltpu.sync_copy(ones_v, acc.at[dstB], add=True)
            return 0
        lax.fori_loop(0, steps // 2, body, 0)
        pltpu.make_async_copy(dst_hbm.at[pl.ds(base0, CHUNK)], dstA,
                              isemA).wait()

        if rem:
            bR = pl.multiple_of(wid * ew + steps * CHUNK, 8)
            pltpu.sync_copy(dst_hbm.at[pl.ds(bR, rem)], dstR)
            pltpu.sync_copy(ones_v.at[pl.ds(0, rem)], acc.at[dstR], add=True)
        if rem2:
            bR2 = pl.multiple_of(wid * ew + steps * CHUNK + rem, 8)
            pltpu.sync_copy(dst_hbm.at[pl.ds(bR2, rem2)], dstR2)
            pltpu.sync_copy(ones_v.at[pl.ds(0, rem2)], acc.at[dstR2],
                            add=True)
        plsc.subcore_barrier()

        pltpu.sync_copy(acc.at[pl.ds(s * rows_t, rows_t)],
                        out_hbm.at[c, pl.ds(s * rows_t, rows_t)])

    return k(dst)[:, :n]


_RB = 2000


def _row_spec(width):
    return pl.BlockSpec((_RB, width), lambda i: (i, 0))


def _part_spec(width):
    return pl.BlockSpec((NC, _RB, width), lambda i: (0, i, 0))


def _full_spec(shape):
    return pl.BlockSpec(shape, lambda i: tuple(0 for _ in shape))


def _tc_prep(degp, x, Wv1):
    n, d = x.shape
    h = Wv1.shape[1]

    def body(degp_ref, x_ref, wv_ref, dinv_ref, xs_ref, hv1_ref):
        deg = degp_ref[0, :, 0:1] + degp_ref[1, :, 0:1] + 1.0
        dinv = lax.rsqrt(deg)
        dinv_ref[...] = dinv
        xs_ref[...] = x_ref[...] * dinv
        hv1_ref[...] = dinv * jnp.dot(x_ref[...], wv_ref[...],
                                      preferred_element_type=jnp.float32)

    return pl.pallas_call(
        body,
        grid=(n // _RB,),
        in_specs=[_part_spec(16), _row_spec(d), _full_spec((d, h))],
        out_specs=[_row_spec(1), _row_spec(d), _row_spec(h)],
        out_shape=[jax.ShapeDtypeStruct((n, 1), jnp.float32),
                   jax.ShapeDtypeStruct((n, d), jnp.float32),
                   jax.ShapeDtypeStruct((n, h), jnp.float32)],
    )(degp, x, Wv1)


def _tc_layer1(p, pv1, xs0, hv1s, dinv, Wp1, bp1, bv1, Wv2):
    n, d = xs0.shape
    h = Wp1.shape[1]

    def body(p_ref, pv1_ref, xs_ref, hv1_ref, dinv_ref, wp_ref, bp_ref,
             bv_ref, wv2_ref, xa_ref, hv2_ref):
        dv = dinv_ref[...]
        z = dv * (p_ref[0] + p_ref[1] + xs_ref[...])
        a1 = jnp.maximum(jnp.dot(z, wp_ref[...],
                                 preferred_element_type=jnp.float32)
                         + bp_ref[...], 0.0)
        xa_ref[...] = dv * a1
        v1 = jnp.maximum(dv * (pv1_ref[0] + pv1_ref[1] + hv1_ref[...])
                         + bv_ref[...], 0.0)
        hv2_ref[...] = dv * jnp.dot(v1, wv2_ref[...],
                                    preferred_element_type=jnp.float32)

    return pl.pallas_call(
        body,
        grid=(n // _RB,),
        in_specs=[_part_spec(d), _part_spec(h), _row_spec(d), _row_spec(h),
                  _row_spec(1),
                  _full_spec((d, h)), _full_spec((1, h)), _full_spec((1, h)),
                  _full_spec((h, h))],
        out_specs=[_row_spec(h), _row_spec(h)],
        out_shape=[jax.ShapeDtypeStruct((n, h), jnp.float32),
                   jax.ShapeDtypeStruct((n, h), jnp.float32)],
    )(p, pv1, xs0, hv1s, dinv, Wp1, bp1.reshape(1, -1), bv1.reshape(1, -1),
      Wv2)


def _tc_layer23(pa, pv2, xa1, hv2s, dinv, Wp2, bp2, Wp3, bv2, Wv3):
    n, h = xa1.shape
    out_p = Wp3.shape[1]

    def body(pa_ref, pv2_ref, xa_ref, hv2_ref, dinv_ref,
             wp2_ref, bp2_ref, wp3_ref, bv2_ref, wv3_ref, hcat_ref):
        dv = dinv_ref[...]
        za = dv * (pa_ref[0] + pa_ref[1] + xa_ref[...])
        a2 = jnp.maximum(jnp.dot(za, wp2_ref[...],
                                 preferred_element_type=jnp.float32)
                         + bp2_ref[...], 0.0)
        hp = jnp.dot(a2, wp3_ref[...], preferred_element_type=jnp.float32)
        v2 = jnp.maximum(dv * (pv2_ref[0] + pv2_ref[1] + hv2_ref[...])
                         + bv2_ref[...], 0.0)
        hv = jnp.dot(v2, wv3_ref[...], preferred_element_type=jnp.float32)
        pad = jnp.zeros((hp.shape[0], 32 - out_p - 1), jnp.float32)
        hcat_ref[...] = dv * jnp.concatenate([hp, hv, pad], axis=1)

    return pl.pallas_call(
        body,
        grid=(n // _RB,),
        in_specs=[_part_spec(h), _part_spec(h), _row_spec(h), _row_spec(h),
                  _row_spec(1),
                  _full_spec((h, h)), _full_spec((1, h)),
                  _full_spec((h, out_p)),
                  _full_spec((1, h)),
                  _full_spec((h, 1))],
        out_specs=[_row_spec(32)],
        out_shape=[jax.ShapeDtypeStruct((n, 32), jnp.float32)],
    )(pa, pv2, xa1, hv2s, dinv, Wp2, bp2.reshape(1, -1), Wp3,
      bv2.reshape(1, -1), Wv3)[0]


def _tc_final(pc, hcat, dinv, bp3, bv3, out_p):
    n = hcat.shape[0]

    def body(pc_ref, hcat_ref, dinv_ref, bp3_ref, bv3_ref, lg_ref, vl_ref):
        cfull = dinv_ref[...] * (pc_ref[0] + pc_ref[1] + hcat_ref[...])
        lg_ref[...] = cfull[:, :out_p] + bp3_ref[...]
        vl_ref[...] = cfull[:, out_p:out_p + 1] + bv3_ref[...]

    return pl.pallas_call(
        body,
        grid=(n // _RB,),
        in_specs=[_part_spec(32), _row_spec(32), _row_spec(1),
                  _full_spec((1, out_p)), _full_spec((1, 1))],
        out_specs=[_row_spec(out_p), _row_spec(1)],
        out_shape=[jax.ShapeDtypeStruct((n, out_p), jnp.float32),
                   jax.ShapeDtypeStruct((n, 1), jnp.float32)],
    )(pc, hcat, dinv, bp3.reshape(1, -1), bv3.reshape(1, -1))


def kernel(x, edge_index, Wp1, bp1, Wp2, bp2, Wp3, bp3, Wv1, bv1, Wv2, bv2,
           Wv3, bv3):
    n, d = x.shape
    out_p = Wp3.shape[1]
    src, dst, ew, steps, rem, rem2 = _pad_edges(edge_index[0], edge_index[1],
                                                n)

    degp = _sc_degree(dst, ew, steps, rem, rem2, n)
    dinv, xs0, hv1s = _tc_prep(degp, x, Wv1)

    p0 = _sc_apply(xs0, src, dst, ew, steps, rem, rem2, n, d)
    pv1 = _sc_apply(hv1s, src, dst, ew, steps, rem, rem2, n, d)
    xa1, hv2s = _tc_layer1(p0, pv1, xs0, hv1s, dinv, Wp1, bp1, bv1, Wv2)

    pa = _sc_apply(xa1, src, dst, ew, steps, rem, rem2, n, d)
    pv2 = _sc_apply(hv2s, src, dst, ew, steps, rem, rem2, n, d)
    hcat = _tc_layer23(pa, pv2, xa1, hv2s, dinv, Wp2, bp2, Wp3, bv2, Wv3)

    pc = _sc_apply(hcat, src, dst, ew, steps, rem, rem2, n, 32)
    logits, value = _tc_final(pc, hcat, dinv, bp3, bv3, out_p)
    return (logits, value)
